# baseline (speedup 1.0000x reference)
import jax
import jax.numpy as jnp
from jax import lax
from jax.experimental import pallas as pl
from jax.experimental.pallas import tpu as pltpu
from jax.experimental.pallas import tpu_sc as plsc

_B, _H, _S, _D = 32, 8, 2048, 64
_RW = 8
_RL = _S
_CH = 512
_NCH = _RL // _CH


def _sc_body(k64, v64, new64, idx_hbm, out64, buf, idx_v, new_v,
             sem_buf, sem_sc):
    wid = lax.axis_index("s") * 2 + lax.axis_index("c")
    pltpu.sync_copy(new64.at[pl.ds(wid * 16, 16)], new_v)
    pltpu.sync_copy(idx_hbm.at[wid], idx_v)

    base = wid * _RW * _RL
    vout = _B * _H * _RL
    n = _RW * _NCH

    def body(c, carry, src, dst_off):
        off = base + c * _CH
        cin = pltpu.make_async_copy(src.at[pl.ds(off, _CH)], buf, sem_buf)
        cin.start()
        cin.wait()
        cout = pltpu.make_async_copy(buf, out64.at[pl.ds(dst_off + off, _CH)],
                                     sem_buf)
        cout.start()
        cout.wait()
        return carry

    lax.fori_loop(0, n, lambda c, a: body(c, a, k64, 0), 0)
    lax.fori_loop(0, n, lambda c, a: body(c, a, v64, vout), 0)
    tgt = idx_v[...]
    for i in range(16):
        line = tgt[i]
        pltpu.sync_copy(new_v.at[pl.ds(i, 1)], out64.at[pl.ds(line, 1)])


def kernel(k_cache, v_cache, new_k, new_v, seq_ids, position_ids):
    b, h, s, d = k_cache.shape
    inv = jnp.argsort(seq_ids).astype(jnp.int32)
    pos = position_ids[inv, 0].astype(jnp.int32)
    k64 = k_cache.reshape(b * h * s, d)
    v64 = v_cache.reshape(b * h * s, d)
    newkv = jnp.concatenate(
        [new_k[inv].reshape(b * h, d), new_v[inv].reshape(b * h, d)], axis=0)
    t = jnp.arange(2 * b * h, dtype=jnp.int32)
    bt = (t % (b * h)) // h
    idx = t * s + pos[bt]
    order = jnp.concatenate(
        [jnp.arange(b * h, dtype=jnp.int32).reshape(32, _RW),
         b * h + jnp.arange(b * h, dtype=jnp.int32).reshape(32, _RW)],
        axis=1).reshape(-1)
    idx3 = idx[order].reshape(32, 16)
    new64 = newkv[order]

    mesh = plsc.VectorSubcoreMesh(core_axis_name="c", subcore_axis_name="s")
    run = pl.kernel(
        _sc_body,
        mesh=mesh,
        out_type=jax.ShapeDtypeStruct((2 * b * h * s, d), k_cache.dtype),
        scratch_types=[
            pltpu.VMEM((_CH, _D), jnp.float32),
            pltpu.VMEM((16,), jnp.int32),
            pltpu.VMEM((16, _D), jnp.float32),
            pltpu.SemaphoreType.DMA,
            pltpu.SemaphoreType.DMA,
        ],
    )
    out = run(k64, v64, new64, idx3)
    return out.reshape(2, b, h, s, d)


# speedup vs baseline: 1.0213x; 1.0213x over previous
import jax
import jax.numpy as jnp
from jax import lax
from jax.experimental import pallas as pl
from jax.experimental.pallas import tpu as pltpu
from jax.experimental.pallas import tpu_sc as plsc

_B, _H, _S, _D = 32, 8, 2048, 64
_RW = 8
_RL = _S
_CH = 256
_NCH = _RL // _CH


def _sc_body(k64, v64, new64, idx_hbm, out64, buf0, buf1, idx_v, new_v,
             sg0, sg1, ss0, ss1):
    wid = lax.axis_index("s") * 2 + lax.axis_index("c")
    pltpu.sync_copy(new64.at[pl.ds(wid * 16, 16)], new_v)
    pltpu.sync_copy(idx_hbm.at[wid], idx_v)

    base = wid * _RW * _RL
    vout = _B * _H * _RL
    n = _RW * _NCH

    bufs = (buf0, buf1)
    gsem = (sg0, sg1)
    ssem = (ss0, ss1)
    pend = [None, None]

    def step(c, src, off, dst_off):
        slot = c % 2
        if pend[slot] is not None:
            pend[slot].wait()
        g = pltpu.make_async_copy(src.at[pl.ds(off, _CH)], bufs[slot],
                                  gsem[slot])
        g.start()
        g.wait()
        w = pltpu.make_async_copy(bufs[slot], out64.at[pl.ds(dst_off, _CH)],
                                  ssem[slot])
        w.start()
        pend[slot] = w

    for c in range(n):
        step(c, k64, base + c * _CH, base + c * _CH)
    for c in range(n):
        step(n + c, v64, base + c * _CH, vout + base + c * _CH)
    for w in pend:
        if w is not None:
            w.wait()
    tgt = idx_v[...]
    for i in range(16):
        line = tgt[i]
        pltpu.sync_copy(new_v.at[pl.ds(i, 1)], out64.at[pl.ds(line, 1)])


def kernel(k_cache, v_cache, new_k, new_v, seq_ids, position_ids):
    b, h, s, d = k_cache.shape
    inv = jnp.argsort(seq_ids).astype(jnp.int32)
    pos = position_ids[inv, 0].astype(jnp.int32)
    k64 = k_cache.reshape(b * h * s, d)
    v64 = v_cache.reshape(b * h * s, d)
    newkv = jnp.concatenate(
        [new_k[inv].reshape(b * h, d), new_v[inv].reshape(b * h, d)], axis=0)
    t = jnp.arange(2 * b * h, dtype=jnp.int32)
    bt = (t % (b * h)) // h
    idx = t * s + pos[bt]
    order = jnp.concatenate(
        [jnp.arange(b * h, dtype=jnp.int32).reshape(32, _RW),
         b * h + jnp.arange(b * h, dtype=jnp.int32).reshape(32, _RW)],
        axis=1).reshape(-1)
    idx3 = idx[order].reshape(32, 16)
    new64 = newkv[order]

    mesh = plsc.VectorSubcoreMesh(core_axis_name="c", subcore_axis_name="s")
    run = pl.kernel(
        _sc_body,
        mesh=mesh,
        out_type=jax.ShapeDtypeStruct((2 * b * h * s, d), k_cache.dtype),
        scratch_types=[
            pltpu.VMEM((_CH, _D), jnp.float32),
            pltpu.VMEM((_CH, _D), jnp.float32),
            pltpu.VMEM((16,), jnp.int32),
            pltpu.VMEM((16, _D), jnp.float32),
            pltpu.SemaphoreType.DMA,
            pltpu.SemaphoreType.DMA,
            pltpu.SemaphoreType.DMA,
            pltpu.SemaphoreType.DMA,
        ],
    )
    out = run(k64, v64, new64, idx3)
    return out.reshape(2, b, h, s, d)
